# baseline (speedup 1.0000x reference)
import functools

import jax
import jax.numpy as jnp
from jax import lax
from jax.experimental import pallas as pl
from jax.experimental.pallas import tpu as pltpu
from jax.experimental.pallas import tpu_sc as plsc

_PREC = lax.Precision.HIGHEST


def _tca_body(x_ref, fmw_ref, fmb_ref, wxi_ref, wxj_ref, oh_ref, nemb_ref,
              wnti_ref, wntj_ref, h_ref, p_ref, q_ref):
    h = jnp.dot(x_ref[...], fmw_ref[...], preferred_element_type=jnp.float32,
                precision=_PREC) + fmb_ref[...]
    h_ref[...] = h
    ti = jnp.dot(nemb_ref[...], wnti_ref[...],
                 preferred_element_type=jnp.float32, precision=_PREC)
    tj = jnp.dot(nemb_ref[...], wntj_ref[...],
                 preferred_element_type=jnp.float32, precision=_PREC)
    oh = oh_ref[...]
    p_ref[...] = (jnp.dot(h, wxi_ref[...], preferred_element_type=jnp.float32,
                          precision=_PREC)
                  + jnp.dot(oh, ti, preferred_element_type=jnp.float32,
                            precision=_PREC))
    q_ref[...] = (jnp.dot(h, wxj_ref[...], preferred_element_type=jnp.float32,
                          precision=_PREC)
                  + jnp.dot(oh, tj, preferred_element_type=jnp.float32,
                            precision=_PREC))


def _tc_a(x, fm_w, fm_b, wxi, wxj, nt_oh, node_emb, wnti, wntj, blk):
    n, f = x.shape
    o = fm_w.shape[1]
    nd = node_emb.shape[1]
    grid = n // blk
    full = lambda i: (0, 0)
    outs = [jax.ShapeDtypeStruct((n, o), jnp.float32)] * 3
    return pl.pallas_call(
        _tca_body,
        grid=(grid,),
        in_specs=[
            pl.BlockSpec((blk, f), lambda i: (i, 0)),
            pl.BlockSpec((f, o), full),
            pl.BlockSpec((1, o), full),
            pl.BlockSpec((o, o), full),
            pl.BlockSpec((o, o), full),
            pl.BlockSpec((blk, 16), lambda i: (i, 0)),
            pl.BlockSpec((16, nd), full),
            pl.BlockSpec((nd, o), full),
            pl.BlockSpec((nd, o), full),
        ],
        out_specs=[pl.BlockSpec((blk, o), lambda i: (i, 0))] * 3,
        out_shape=outs,
    )(x, fm_w, fm_b.reshape(1, o), wxi, wxj, nt_oh, node_emb, wnti, wntj)


def _tcb_body(g1_ref, g2_ref, g3_ref, eoh_ref, ew_ref, eemb_ref, wet_ref,
              g1b_ref, g2w_ref, g2b_ref, out_ref):
    et_tab = jnp.dot(eemb_ref[...], wet_ref[...],
                     preferred_element_type=jnp.float32,
                     precision=_PREC) + g1b_ref[...]
    u = g1_ref[...] + g2_ref[...] + jnp.dot(
        eoh_ref[...], et_tab, preferred_element_type=jnp.float32,
        precision=_PREC)
    u = jnp.maximum(u, 0.0)
    t = jnp.sum(u * g2w_ref[...], axis=1, keepdims=True) + g2b_ref[...]
    s = jnp.maximum(t, 0.0) * ew_ref[...]
    m = g3_ref[...] * s
    half = m.shape[1] // 2
    out_ref[0] = m[:, :half]
    out_ref[1] = m[:, half:]


def _tc_b(g1, g2, g3, et_oh, ew, edge_emb, wet, g1b, g2w, g2b, blk):
    e, o = g1.shape
    ed = edge_emb.shape[1]
    grid = e // blk
    full = lambda i: (0, 0)
    return pl.pallas_call(
        _tcb_body,
        grid=(grid,),
        in_specs=[
            pl.BlockSpec((blk, o), lambda i: (i, 0)),
            pl.BlockSpec((blk, o), lambda i: (i, 0)),
            pl.BlockSpec((blk, o), lambda i: (i, 0)),
            pl.BlockSpec((blk, 16), lambda i: (i, 0)),
            pl.BlockSpec((blk, 1), lambda i: (i, 0)),
            pl.BlockSpec((16, ed), full),
            pl.BlockSpec((ed, o), full),
            pl.BlockSpec((1, o), full),
            pl.BlockSpec((1, o), full),
            pl.BlockSpec((1, 1), full),
        ],
        out_specs=[pl.BlockSpec((2, blk, o // 2), lambda i: (0, i, 0))],
        out_shape=[jax.ShapeDtypeStruct((2, e, o // 2), jnp.float32)],
    )(g1, g2, g3, et_oh, ew, edge_emb, wet, g1b.reshape(1, o),
      g2w.reshape(1, o), g2b.reshape(1, 1))[0]


def _tcca_body(hp_ref, a_ref, fmw_ref, fmb_ref, wxi_ref, wxj_ref, oh_ref,
               nemb_ref, wnti_ref, wntj_ref, h_ref, p_ref, q_ref):
    half = hp_ref.shape[1] // 2
    v0 = hp_ref[:, :half] + a_ref[0]
    v1 = hp_ref[:, half:] + a_ref[1]
    x = jnp.concatenate([jnp.where(v0 >= 0.0, v0, 0.01 * v0),
                         jnp.where(v1 >= 0.0, v1, 0.01 * v1)], axis=1)
    h = jnp.dot(x, fmw_ref[...], preferred_element_type=jnp.float32,
                precision=_PREC) + fmb_ref[...]
    h_ref[...] = h
    ti = jnp.dot(nemb_ref[...], wnti_ref[...],
                 preferred_element_type=jnp.float32, precision=_PREC)
    tj = jnp.dot(nemb_ref[...], wntj_ref[...],
                 preferred_element_type=jnp.float32, precision=_PREC)
    oh = oh_ref[...]
    p_ref[...] = (jnp.dot(h, wxi_ref[...], preferred_element_type=jnp.float32,
                          precision=_PREC)
                  + jnp.dot(oh, ti, preferred_element_type=jnp.float32,
                            precision=_PREC))
    q_ref[...] = (jnp.dot(h, wxj_ref[...], preferred_element_type=jnp.float32,
                          precision=_PREC)
                  + jnp.dot(oh, tj, preferred_element_type=jnp.float32,
                            precision=_PREC))


def _tc_ca(h_prev, aggr, fm_w, fm_b, wxi, wxj, nt_oh, node_emb, wnti, wntj,
           blk):
    n, f = h_prev.shape
    o = fm_w.shape[1]
    nd = node_emb.shape[1]
    grid = n // blk
    full = lambda i: (0, 0)
    outs = [jax.ShapeDtypeStruct((n, o), jnp.float32)] * 3
    return pl.pallas_call(
        _tcca_body,
        grid=(grid,),
        in_specs=[
            pl.BlockSpec((blk, f), lambda i: (i, 0)),
            pl.BlockSpec((2, blk, f // 2), lambda i: (0, i, 0)),
            pl.BlockSpec((f, o), full),
            pl.BlockSpec((1, o), full),
            pl.BlockSpec((o, o), full),
            pl.BlockSpec((o, o), full),
            pl.BlockSpec((blk, 16), lambda i: (i, 0)),
            pl.BlockSpec((16, nd), full),
            pl.BlockSpec((nd, o), full),
            pl.BlockSpec((nd, o), full),
        ],
        out_specs=[pl.BlockSpec((blk, o), lambda i: (i, 0))] * 3,
        out_shape=outs,
    )(h_prev, aggr, fm_w, fm_b.reshape(1, o), wxi, wxj, nt_oh, node_emb,
      wnti, wntj)


def _tcc_body(h_ref, a_ref, out_ref):
    half = h_ref.shape[1] // 2
    v0 = h_ref[:, :half] + a_ref[0]
    v1 = h_ref[:, half:] + a_ref[1]
    out_ref[:, :half] = jnp.where(v0 >= 0.0, v0, 0.01 * v0)
    out_ref[:, half:] = jnp.where(v1 >= 0.0, v1, 0.01 * v1)


def _tc_c(h, aggr, blk):
    n, o = h.shape
    grid = n // blk
    return pl.pallas_call(
        _tcc_body,
        grid=(grid,),
        in_specs=[
            pl.BlockSpec((blk, o), lambda i: (i, 0)),
            pl.BlockSpec((2, blk, o // 2), lambda i: (0, i, 0)),
        ],
        out_specs=pl.BlockSpec((blk, o), lambda i: (i, 0)),
        out_shape=jax.ShapeDtypeStruct((n, o), jnp.float32),
    )(h, aggr)


def _sc_gather(p, q, h, dst2, src2):
    n, d = p.shape
    e = dst2.shape[1]
    w = 128
    nc = 2
    steps_per_core = e // w // nc
    mesh = plsc.VectorSubcoreMesh(core_axis_name="c", subcore_axis_name="s")
    outs = [jax.ShapeDtypeStruct((e, d), jnp.float32)] * 3

    @functools.partial(pl.kernel, mesh=mesh, out_type=outs, scratch_types=[])
    def k(p_hbm, q_hbm, h_hbm, dst_hbm, src_hbm, g1_hbm, g2_hbm, g3_hbm):
        idx_spec = pl.BlockSpec((1, w), lambda c, j: (0, c * steps_per_core + j))
        row_spec = pl.BlockSpec((w, d), lambda c, j: (c * steps_per_core + j, 0))

        def one_table(tbl_hbm, ihbm, ohbm):
            def body(i_v, o_v):
                pltpu.sync_copy(tbl_hbm.at[i_v.at[0]], o_v)

            pltpu.emit_pipeline(
                body,
                grid=(nc, steps_per_core),
                in_specs=[idx_spec],
                out_specs=[row_spec],
                core_axis_name=("c", "s"),
                dimension_semantics=(pltpu.PARALLEL, pltpu.PARALLEL),
            )(ihbm, ohbm)

        one_table(p_hbm, dst_hbm, g1_hbm)
        one_table(q_hbm, src_hbm, g2_hbm)
        one_table(h_hbm, src_hbm, g3_hbm)

    return k(p, q, h, dst2, src2)


def _sc_scatter(msg_a, msg_b, dst_a, dst_b, zeros_half):
    _, ea, d2 = msg_a.shape
    eb = msg_b.shape[1]
    n = zeros_half.shape[0]
    w = 128
    ns = 16
    rows = n // ns
    r_lo = (rows // 8) * 8
    r_hi = n - r_lo * (ns - 1)
    mesh = plsc.VectorSubcoreMesh(core_axis_name="c", subcore_axis_name="s")

    @functools.partial(
        pl.kernel, mesh=mesh,
        out_type=jax.ShapeDtypeStruct((2, n, d2), jnp.float32),
        scratch_types=[pltpu.VMEM_SHARED((n, d2), jnp.float32)])
    def k(msga_hbm, msgb_hbm, dsta_hbm, dstb_hbm, z_hbm, out_hbm, aggr_sh):
        cid = lax.axis_index("c")
        sid = lax.axis_index("s")

        @pl.when(sid < ns - 1)
        def _():
            pltpu.sync_copy(z_hbm.at[pl.ds(sid * r_lo, r_lo)],
                            aggr_sh.at[pl.ds(sid * r_lo, r_lo)])

        @pl.when(sid == ns - 1)
        def _():
            pltpu.sync_copy(z_hbm.at[pl.ds((ns - 1) * r_lo, r_hi)],
                            aggr_sh.at[pl.ds((ns - 1) * r_lo, r_hi)])

        plsc.subcore_barrier()

        def body(msg_v, idx_v):
            pltpu.sync_copy(msg_v, aggr_sh.at[idx_v.at[0]], add=True)

        for m_hbm, i_hbm, ee in ((msga_hbm, dsta_hbm, ea),
                                 (msgb_hbm, dstb_hbm, eb)):
            pltpu.emit_pipeline(
                body,
                grid=(ee // w,),
                in_specs=[
                    pl.BlockSpec((w, d2), lambda i: (i, 0)),
                    pl.BlockSpec((1, w), lambda i: (0, i)),
                ],
                out_specs=[],
                core_axis_name="s",
                dimension_semantics=(pltpu.PARALLEL,),
            )(m_hbm.at[cid], i_hbm)

        plsc.subcore_barrier()

        @pl.when(sid < ns - 1)
        def _():
            pltpu.sync_copy(aggr_sh.at[pl.ds(sid * r_lo, r_lo)],
                            out_hbm.at[cid].at[pl.ds(sid * r_lo, r_lo)])

        @pl.when(sid == ns - 1)
        def _():
            pltpu.sync_copy(aggr_sh.at[pl.ds((ns - 1) * r_lo, r_hi)],
                            out_hbm.at[cid].at[pl.ds((ns - 1) * r_lo, r_hi)])

    return k(msg_a, msg_b, dst_a, dst_b, zeros_half)


def kernel(x, edge_index, edge_attr, node_type, edge_type, node_emb, edge_emb,
           fm_w0, fm_b0, g1_w0, g1_b0, g2_w0, g2_b0,
           fm_w1, fm_b1, g1_w1, g1_b1, g2_w1, g2_b1):
    n, f = x.shape[1], x.shape[2]
    e = edge_index.shape[1]
    o = fm_w0.shape[1]
    nd = node_emb.shape[1]

    xs = x.reshape(n, f)
    src = edge_index[0]
    dst = edge_index[1]
    nt_oh = jax.nn.one_hot(node_type, 16, dtype=jnp.float32)
    et_oh = jax.nn.one_hot(edge_type, 16, dtype=jnp.float32)
    ew = edge_attr.reshape(e, 1)
    dst2 = dst.reshape(1, e)
    src2 = src.reshape(1, e)
    zeros_half = jnp.zeros((n, o // 2), jnp.float32)

    ea = 81920
    chunks = [(0, ea), (ea, e - ea)]

    def split(arr_rows, c0, ec):
        return lax.slice_in_dim(arr_rows, c0, c0 + ec, axis=0)

    def layer(h, p, q, g1b, g2w, g2b, wet):
        gs = [_sc_gather(p, q, h, dst2[:, c0:c0 + ec], src2[:, c0:c0 + ec])
              for c0, ec in chunks]
        msgs = [_tc_b(g1, g2, g3, split(et_oh, c0, ec), split(ew, c0, ec),
                      edge_emb, wet, g1b, g2w, g2b, blk=1280)
                for (g1, g2, g3), (c0, ec) in zip(gs, chunks)]
        return _sc_scatter(msgs[0], msgs[1], dst2[:, :ea], dst2[:, ea:],
                           zeros_half)

    def wparts(g1w):
        return (g1w[0:o], g1w[o:2 * o], g1w[2 * o:2 * o + nd],
                g1w[2 * o + nd:2 * o + 2 * nd], g1w[2 * o + 2 * nd:])

    wxi0, wxj0, wnti0, wntj0, wet0 = wparts(g1_w0)
    wxi1, wxj1, wnti1, wntj1, wet1 = wparts(g1_w1)

    h1, p1, q1 = _tc_a(xs, fm_w0, fm_b0, wxi0, wxj0, nt_oh, node_emb,
                       wnti0, wntj0, blk=1000)
    aggr1 = layer(h1, p1, q1, g1_b0, g2_w0, g2_b0, wet0)
    h2, p2, q2 = _tc_ca(h1, aggr1, fm_w1, fm_b1, wxi1, wxj1, nt_oh,
                        node_emb, wnti1, wntj1, blk=1000)
    aggr2 = layer(h2, p2, q2, g1_b1, g2_w1, g2_b1, wet1)
    out = _tc_c(h2, aggr2, blk=1000)
    return out.reshape(1, n, o)


# speedup vs baseline: 1.1876x; 1.1876x over previous
import functools

import jax
import jax.numpy as jnp
from jax import lax
from jax.experimental import pallas as pl
from jax.experimental.pallas import tpu as pltpu
from jax.experimental.pallas import tpu_sc as plsc

_PREC = lax.Precision.HIGHEST


def _pack(v):
    half = v.shape[1] // 2
    lo = v[:, :half].astype(jnp.bfloat16).astype(jnp.float32)
    hi = v[:, half:].astype(jnp.bfloat16).astype(jnp.float32)
    lo_u = jax.lax.bitcast_convert_type(lo, jnp.uint32)
    hi_u = jax.lax.bitcast_convert_type(hi, jnp.uint32)
    return jax.lax.bitcast_convert_type(hi_u | (lo_u >> 16), jnp.float32)


def _unpack(w):
    w_u = jax.lax.bitcast_convert_type(w, jnp.uint32)
    lo = jax.lax.bitcast_convert_type(w_u << 16, jnp.float32)
    hi = jax.lax.bitcast_convert_type(w_u & jnp.uint32(0xFFFF0000),
                                      jnp.float32)
    return lo, hi


def _node_terms(h, oh, nemb_ref, wxi_ref, wxj_ref, wnti_ref, wntj_ref):
    ti = jnp.dot(nemb_ref[...], wnti_ref[...],
                 preferred_element_type=jnp.float32, precision=_PREC)
    tj = jnp.dot(nemb_ref[...], wntj_ref[...],
                 preferred_element_type=jnp.float32, precision=_PREC)
    p = (jnp.dot(h, wxi_ref[...], preferred_element_type=jnp.float32,
                 precision=_PREC)
         + jnp.dot(oh, ti, preferred_element_type=jnp.float32,
                   precision=_PREC))
    q = (jnp.dot(h, wxj_ref[...], preferred_element_type=jnp.float32,
                 precision=_PREC)
         + jnp.dot(oh, tj, preferred_element_type=jnp.float32,
                   precision=_PREC))
    return p, q


def _tca_body(x_ref, fmw_ref, fmb_ref, wxi_ref, wxj_ref, oh_ref, nemb_ref,
              wnti_ref, wntj_ref, h_ref, pp_ref, qp_ref, hp_ref):
    h = jnp.dot(x_ref[...], fmw_ref[...], preferred_element_type=jnp.float32,
                precision=_PREC) + fmb_ref[...]
    h_ref[...] = h
    p, q = _node_terms(h, oh_ref[...], nemb_ref, wxi_ref, wxj_ref,
                       wnti_ref, wntj_ref)
    pp_ref[...] = _pack(p)
    qp_ref[...] = _pack(q)
    hp_ref[...] = _pack(h)


def _tc_a(x, fm_w, fm_b, wxi, wxj, nt_oh, node_emb, wnti, wntj, blk):
    n, f = x.shape
    o = fm_w.shape[1]
    nd = node_emb.shape[1]
    grid = n // blk
    full = lambda i: (0, 0)
    outs = ([jax.ShapeDtypeStruct((n, o), jnp.float32)]
            + [jax.ShapeDtypeStruct((n, o // 2), jnp.float32)] * 3)
    return pl.pallas_call(
        _tca_body,
        grid=(grid,),
        in_specs=[
            pl.BlockSpec((blk, f), lambda i: (i, 0)),
            pl.BlockSpec((f, o), full),
            pl.BlockSpec((1, o), full),
            pl.BlockSpec((o, o), full),
            pl.BlockSpec((o, o), full),
            pl.BlockSpec((blk, 16), lambda i: (i, 0)),
            pl.BlockSpec((16, nd), full),
            pl.BlockSpec((nd, o), full),
            pl.BlockSpec((nd, o), full),
        ],
        out_specs=([pl.BlockSpec((blk, o), lambda i: (i, 0))]
                   + [pl.BlockSpec((blk, o // 2), lambda i: (i, 0))] * 3),
        out_shape=outs,
    )(x, fm_w, fm_b.reshape(1, o), wxi, wxj, nt_oh, node_emb, wnti, wntj)


def _tcca_body(hprev_ref, a_ref, fmw_ref, fmb_ref, wxi_ref, wxj_ref, oh_ref,
               nemb_ref, wnti_ref, wntj_ref, h_ref, pp_ref, qp_ref, hp_ref):
    half = hprev_ref.shape[1] // 2
    v0 = hprev_ref[:, :half] + a_ref[0]
    v1 = hprev_ref[:, half:] + a_ref[1]
    x = jnp.concatenate([jnp.where(v0 >= 0.0, v0, 0.01 * v0),
                         jnp.where(v1 >= 0.0, v1, 0.01 * v1)], axis=1)
    h = jnp.dot(x, fmw_ref[...], preferred_element_type=jnp.float32,
                precision=_PREC) + fmb_ref[...]
    h_ref[...] = h
    p, q = _node_terms(h, oh_ref[...], nemb_ref, wxi_ref, wxj_ref,
                       wnti_ref, wntj_ref)
    pp_ref[...] = _pack(p)
    qp_ref[...] = _pack(q)
    hp_ref[...] = _pack(h)


def _tc_ca(h_prev, aggr, fm_w, fm_b, wxi, wxj, nt_oh, node_emb, wnti, wntj,
           blk):
    n, f = h_prev.shape
    o = fm_w.shape[1]
    nd = node_emb.shape[1]
    grid = n // blk
    full = lambda i: (0, 0)
    outs = ([jax.ShapeDtypeStruct((n, o), jnp.float32)]
            + [jax.ShapeDtypeStruct((n, o // 2), jnp.float32)] * 3)
    return pl.pallas_call(
        _tcca_body,
        grid=(grid,),
        in_specs=[
            pl.BlockSpec((blk, f), lambda i: (i, 0)),
            pl.BlockSpec((2, blk, f // 2), lambda i: (0, i, 0)),
            pl.BlockSpec((f, o), full),
            pl.BlockSpec((1, o), full),
            pl.BlockSpec((o, o), full),
            pl.BlockSpec((o, o), full),
            pl.BlockSpec((blk, 16), lambda i: (i, 0)),
            pl.BlockSpec((16, nd), full),
            pl.BlockSpec((nd, o), full),
            pl.BlockSpec((nd, o), full),
        ],
        out_specs=([pl.BlockSpec((blk, o), lambda i: (i, 0))]
                   + [pl.BlockSpec((blk, o // 2), lambda i: (i, 0))] * 3),
        out_shape=outs,
    )(h_prev, aggr, fm_w, fm_b.reshape(1, o), wxi, wxj, nt_oh, node_emb,
      wnti, wntj)


def _tcb_body(g1_ref, g2_ref, g3_ref, eoh_ref, ew_ref, eemb_ref, wet_ref,
              g1b_ref, g2w_ref, g2b_ref, out_ref):
    half = g2w_ref.shape[1] // 2
    et_tab = jnp.dot(eemb_ref[...], wet_ref[...],
                     preferred_element_type=jnp.float32,
                     precision=_PREC) + g1b_ref[...]
    etv = jnp.dot(eoh_ref[...], et_tab, preferred_element_type=jnp.float32,
                  precision=_PREC)
    g1_lo, g1_hi = _unpack(g1_ref[...])
    g2_lo, g2_hi = _unpack(g2_ref[...])
    u_lo = jnp.maximum(g1_lo + g2_lo + etv[:, :half], 0.0)
    u_hi = jnp.maximum(g1_hi + g2_hi + etv[:, half:], 0.0)
    t = (jnp.sum(u_lo * g2w_ref[:, :half], axis=1, keepdims=True)
         + jnp.sum(u_hi * g2w_ref[:, half:], axis=1, keepdims=True)
         + g2b_ref[...])
    s = jnp.maximum(t, 0.0) * ew_ref[...]
    g3_lo, g3_hi = _unpack(g3_ref[...])
    out_ref[0] = g3_lo * s
    out_ref[1] = g3_hi * s


def _tc_b(g1, g2, g3, et_oh, ew, edge_emb, wet, g1b, g2w, g2b, blk):
    e, d2 = g1.shape
    o = 2 * d2
    ed = edge_emb.shape[1]
    grid = e // blk
    full = lambda i: (0, 0)
    return pl.pallas_call(
        _tcb_body,
        grid=(grid,),
        in_specs=[
            pl.BlockSpec((blk, d2), lambda i: (i, 0)),
            pl.BlockSpec((blk, d2), lambda i: (i, 0)),
            pl.BlockSpec((blk, d2), lambda i: (i, 0)),
            pl.BlockSpec((blk, 16), lambda i: (i, 0)),
            pl.BlockSpec((blk, 1), lambda i: (i, 0)),
            pl.BlockSpec((16, ed), full),
            pl.BlockSpec((ed, o), full),
            pl.BlockSpec((1, o), full),
            pl.BlockSpec((1, o), full),
            pl.BlockSpec((1, 1), full),
        ],
        out_specs=[pl.BlockSpec((2, blk, d2), lambda i: (0, i, 0))],
        out_shape=[jax.ShapeDtypeStruct((2, e, d2), jnp.float32)],
    )(g1, g2, g3, et_oh, ew, edge_emb, wet, g1b.reshape(1, o),
      g2w.reshape(1, o), g2b.reshape(1, 1))[0]


def _tcc_body(h_ref, a_ref, out_ref):
    half = h_ref.shape[1] // 2
    v0 = h_ref[:, :half] + a_ref[0]
    v1 = h_ref[:, half:] + a_ref[1]
    out_ref[:, :half] = jnp.where(v0 >= 0.0, v0, 0.01 * v0)
    out_ref[:, half:] = jnp.where(v1 >= 0.0, v1, 0.01 * v1)


def _tc_c(h, aggr, blk):
    n, o = h.shape
    grid = n // blk
    return pl.pallas_call(
        _tcc_body,
        grid=(grid,),
        in_specs=[
            pl.BlockSpec((blk, o), lambda i: (i, 0)),
            pl.BlockSpec((2, blk, o // 2), lambda i: (0, i, 0)),
        ],
        out_specs=pl.BlockSpec((blk, o), lambda i: (i, 0)),
        out_shape=jax.ShapeDtypeStruct((n, o), jnp.float32),
    )(h, aggr)


def _sc_gather(pp, qp, hp, dst2, src2):
    n, d = pp.shape
    e = dst2.shape[1]
    w = 128
    nc = 2
    steps_per_core = e // w // nc
    mesh = plsc.VectorSubcoreMesh(core_axis_name="c", subcore_axis_name="s")
    outs = [jax.ShapeDtypeStruct((e, d), jnp.float32)] * 3

    @functools.partial(pl.kernel, mesh=mesh, out_type=outs, scratch_types=[])
    def k(p_hbm, q_hbm, h_hbm, dst_hbm, src_hbm, g1_hbm, g2_hbm, g3_hbm):
        idx_spec = pl.BlockSpec((1, w), lambda c, j: (0, c * steps_per_core + j))
        row_spec = pl.BlockSpec((w, d), lambda c, j: (c * steps_per_core + j, 0))

        def body(di_v, si_v, o1_v, o2_v, o3_v):
            pltpu.sync_copy(p_hbm.at[di_v.at[0]], o1_v)
            pltpu.sync_copy(q_hbm.at[si_v.at[0]], o2_v)
            pltpu.sync_copy(h_hbm.at[si_v.at[0]], o3_v)

        pltpu.emit_pipeline(
            body,
            grid=(nc, steps_per_core),
            in_specs=[idx_spec, idx_spec],
            out_specs=[row_spec] * 3,
            core_axis_name=("c", "s"),
            dimension_semantics=(pltpu.PARALLEL, pltpu.PARALLEL),
        )(dst_hbm, src_hbm, g1_hbm, g2_hbm, g3_hbm)

    return k(pp, qp, hp, dst2, src2)


def _sc_scatter(msg, dst2, zeros_half):
    _, e, d2 = msg.shape
    n = zeros_half.shape[0]
    w = 128
    steps = e // w
    ns = 16
    rows = n // ns
    r_lo = (rows // 8) * 8
    r_hi = n - r_lo * (ns - 1)
    mesh = plsc.VectorSubcoreMesh(core_axis_name="c", subcore_axis_name="s")

    @functools.partial(
        pl.kernel, mesh=mesh,
        out_type=jax.ShapeDtypeStruct((2, n, d2), jnp.float32),
        scratch_types=[pltpu.VMEM_SHARED((n, d2), jnp.float32)])
    def k(msg_hbm, dst_hbm, z_hbm, out_hbm, aggr_sh):
        cid = lax.axis_index("c")
        sid = lax.axis_index("s")

        @pl.when(sid < ns - 1)
        def _():
            pltpu.sync_copy(z_hbm.at[pl.ds(sid * r_lo, r_lo)],
                            aggr_sh.at[pl.ds(sid * r_lo, r_lo)])

        @pl.when(sid == ns - 1)
        def _():
            pltpu.sync_copy(z_hbm.at[pl.ds((ns - 1) * r_lo, r_hi)],
                            aggr_sh.at[pl.ds((ns - 1) * r_lo, r_hi)])

        plsc.subcore_barrier()

        def body(msg_v, idx_v):
            pltpu.sync_copy(msg_v, aggr_sh.at[idx_v.at[0]], add=True)

        pltpu.emit_pipeline(
            body,
            grid=(steps,),
            in_specs=[
                pl.BlockSpec((w, d2), lambda i: (i, 0)),
                pl.BlockSpec((1, w), lambda i: (0, i)),
            ],
            out_specs=[],
            core_axis_name="s",
            dimension_semantics=(pltpu.PARALLEL,),
        )(msg_hbm.at[cid], dst_hbm)

        plsc.subcore_barrier()

        @pl.when(sid < ns - 1)
        def _():
            pltpu.sync_copy(aggr_sh.at[pl.ds(sid * r_lo, r_lo)],
                            out_hbm.at[cid].at[pl.ds(sid * r_lo, r_lo)])

        @pl.when(sid == ns - 1)
        def _():
            pltpu.sync_copy(aggr_sh.at[pl.ds((ns - 1) * r_lo, r_hi)],
                            out_hbm.at[cid].at[pl.ds((ns - 1) * r_lo, r_hi)])

    return k(msg, dst2, zeros_half)


def kernel(x, edge_index, edge_attr, node_type, edge_type, node_emb, edge_emb,
           fm_w0, fm_b0, g1_w0, g1_b0, g2_w0, g2_b0,
           fm_w1, fm_b1, g1_w1, g1_b1, g2_w1, g2_b1):
    n, f = x.shape[1], x.shape[2]
    e = edge_index.shape[1]
    o = fm_w0.shape[1]
    nd = node_emb.shape[1]

    xs = x.reshape(n, f)
    src = edge_index[0]
    dst = edge_index[1]
    nt_oh = jax.nn.one_hot(node_type, 16, dtype=jnp.float32)
    et_oh = jax.nn.one_hot(edge_type, 16, dtype=jnp.float32)
    ew = edge_attr.reshape(e, 1)
    dst2 = dst.reshape(1, e)
    src2 = src.reshape(1, e)
    zeros_half = jnp.zeros((n, o // 2), jnp.float32)

    def layer(h, pp, qp, hp, g1b, g2w, g2b, wet):
        g1, g2, g3 = _sc_gather(pp, qp, hp, dst2, src2)
        msg = _tc_b(g1, g2, g3, et_oh, ew, edge_emb, wet, g1b, g2w, g2b,
                    blk=2000)
        return _sc_scatter(msg, dst2, zeros_half)

    def wparts(g1w):
        return (g1w[0:o], g1w[o:2 * o], g1w[2 * o:2 * o + nd],
                g1w[2 * o + nd:2 * o + 2 * nd], g1w[2 * o + 2 * nd:])

    wxi0, wxj0, wnti0, wntj0, wet0 = wparts(g1_w0)
    wxi1, wxj1, wnti1, wntj1, wet1 = wparts(g1_w1)

    h1, pp1, qp1, hp1 = _tc_a(xs, fm_w0, fm_b0, wxi0, wxj0, nt_oh, node_emb,
                              wnti0, wntj0, blk=1000)
    aggr1 = layer(h1, pp1, qp1, hp1, g1_b0, g2_w0, g2_b0, wet0)
    h2, pp2, qp2, hp2 = _tc_ca(h1, aggr1, fm_w1, fm_b1, wxi1, wxj1, nt_oh,
                               node_emb, wnti1, wntj1, blk=1000)
    aggr2 = layer(h2, pp2, qp2, hp2, g1_b1, g2_w1, g2_b1, wet1)
    out = _tc_c(h2, aggr2, blk=1000)
    return out.reshape(1, n, o)


# speedup vs baseline: 1.2259x; 1.0323x over previous
import functools

import jax
import jax.numpy as jnp
from jax import lax
from jax.experimental import pallas as pl
from jax.experimental.pallas import tpu as pltpu
from jax.experimental.pallas import tpu_sc as plsc

_PREC = lax.Precision.HIGHEST


def _pack(v):
    half = v.shape[1] // 2
    lo = v[:, :half].astype(jnp.bfloat16).astype(jnp.float32)
    hi = v[:, half:].astype(jnp.bfloat16).astype(jnp.float32)
    lo_u = jax.lax.bitcast_convert_type(lo, jnp.uint32)
    hi_u = jax.lax.bitcast_convert_type(hi, jnp.uint32)
    return jax.lax.bitcast_convert_type(hi_u | (lo_u >> 16), jnp.float32)


def _unpack(w):
    w_u = jax.lax.bitcast_convert_type(w, jnp.uint32)
    lo = jax.lax.bitcast_convert_type(w_u << 16, jnp.float32)
    hi = jax.lax.bitcast_convert_type(w_u & jnp.uint32(0xFFFF0000),
                                      jnp.float32)
    return lo, hi


def _node_terms(h, oh, nemb_ref, wxi_ref, wxj_ref, wnti_ref, wntj_ref):
    ti = jnp.dot(nemb_ref[...], wnti_ref[...],
                 preferred_element_type=jnp.float32, precision=_PREC)
    tj = jnp.dot(nemb_ref[...], wntj_ref[...],
                 preferred_element_type=jnp.float32, precision=_PREC)
    p = (jnp.dot(h, wxi_ref[...], preferred_element_type=jnp.float32,
                 precision=_PREC)
         + jnp.dot(oh, ti, preferred_element_type=jnp.float32,
                   precision=_PREC))
    q = (jnp.dot(h, wxj_ref[...], preferred_element_type=jnp.float32,
                 precision=_PREC)
         + jnp.dot(oh, tj, preferred_element_type=jnp.float32,
                   precision=_PREC))
    return p, q


def _tca_body(x_ref, fmw_ref, fmb_ref, wxi_ref, wxj_ref, oh_ref, nemb_ref,
              wnti_ref, wntj_ref, h_ref, pp_ref, qp_ref, hp_ref):
    h = jnp.dot(x_ref[...], fmw_ref[...], preferred_element_type=jnp.float32,
                precision=_PREC) + fmb_ref[...]
    h_ref[...] = h
    p, q = _node_terms(h, oh_ref[...], nemb_ref, wxi_ref, wxj_ref,
                       wnti_ref, wntj_ref)
    pp_ref[...] = _pack(p)
    qp_ref[...] = _pack(q)
    hp_ref[...] = _pack(h)


def _tc_a(x, fm_w, fm_b, wxi, wxj, nt_oh, node_emb, wnti, wntj, blk):
    n, f = x.shape
    o = fm_w.shape[1]
    nd = node_emb.shape[1]
    grid = n // blk
    full = lambda i: (0, 0)
    outs = ([jax.ShapeDtypeStruct((n, o), jnp.float32)]
            + [jax.ShapeDtypeStruct((n, o // 2), jnp.float32)] * 3)
    return pl.pallas_call(
        _tca_body,
        grid=(grid,),
        in_specs=[
            pl.BlockSpec((blk, f), lambda i: (i, 0)),
            pl.BlockSpec((f, o), full),
            pl.BlockSpec((1, o), full),
            pl.BlockSpec((o, o), full),
            pl.BlockSpec((o, o), full),
            pl.BlockSpec((blk, 16), lambda i: (i, 0)),
            pl.BlockSpec((16, nd), full),
            pl.BlockSpec((nd, o), full),
            pl.BlockSpec((nd, o), full),
        ],
        out_specs=([pl.BlockSpec((blk, o), lambda i: (i, 0))]
                   + [pl.BlockSpec((blk, o // 2), lambda i: (i, 0))] * 3),
        out_shape=outs,
    )(x, fm_w, fm_b.reshape(1, o), wxi, wxj, nt_oh, node_emb, wnti, wntj)


def _tcca_body(hprev_ref, a_ref, fmw_ref, fmb_ref, wxi_ref, wxj_ref, oh_ref,
               nemb_ref, wnti_ref, wntj_ref, h_ref, pp_ref, qp_ref, hp_ref):
    half = hprev_ref.shape[1] // 2
    v0 = hprev_ref[:, :half] + a_ref[0]
    v1 = hprev_ref[:, half:] + a_ref[1]
    x = jnp.concatenate([jnp.where(v0 >= 0.0, v0, 0.01 * v0),
                         jnp.where(v1 >= 0.0, v1, 0.01 * v1)], axis=1)
    h = jnp.dot(x, fmw_ref[...], preferred_element_type=jnp.float32,
                precision=_PREC) + fmb_ref[...]
    h_ref[...] = h
    p, q = _node_terms(h, oh_ref[...], nemb_ref, wxi_ref, wxj_ref,
                       wnti_ref, wntj_ref)
    pp_ref[...] = _pack(p)
    qp_ref[...] = _pack(q)
    hp_ref[...] = _pack(h)


def _tc_ca(h_prev, aggr, fm_w, fm_b, wxi, wxj, nt_oh, node_emb, wnti, wntj,
           blk):
    n, f = h_prev.shape
    o = fm_w.shape[1]
    nd = node_emb.shape[1]
    grid = n // blk
    full = lambda i: (0, 0)
    outs = ([jax.ShapeDtypeStruct((n, o), jnp.float32)]
            + [jax.ShapeDtypeStruct((n, o // 2), jnp.float32)] * 3)
    return pl.pallas_call(
        _tcca_body,
        grid=(grid,),
        in_specs=[
            pl.BlockSpec((blk, f), lambda i: (i, 0)),
            pl.BlockSpec((2, blk, f // 2), lambda i: (0, i, 0)),
            pl.BlockSpec((f, o), full),
            pl.BlockSpec((1, o), full),
            pl.BlockSpec((o, o), full),
            pl.BlockSpec((o, o), full),
            pl.BlockSpec((blk, 16), lambda i: (i, 0)),
            pl.BlockSpec((16, nd), full),
            pl.BlockSpec((nd, o), full),
            pl.BlockSpec((nd, o), full),
        ],
        out_specs=([pl.BlockSpec((blk, o), lambda i: (i, 0))]
                   + [pl.BlockSpec((blk, o // 2), lambda i: (i, 0))] * 3),
        out_shape=outs,
    )(h_prev, aggr, fm_w, fm_b.reshape(1, o), wxi, wxj, nt_oh, node_emb,
      wnti, wntj)


def _tcb_body(g1_ref, g2_ref, g3_ref, eoh_ref, ew_ref, eemb_ref, wet_ref,
              g1b_ref, g2w_ref, g2b_ref, out_ref):
    half = g2w_ref.shape[1] // 2
    et_tab = jnp.dot(eemb_ref[...], wet_ref[...],
                     preferred_element_type=jnp.float32,
                     precision=_PREC) + g1b_ref[...]
    etv = jnp.dot(eoh_ref[...], et_tab, preferred_element_type=jnp.float32,
                  precision=_PREC)
    g1_lo, g1_hi = _unpack(g1_ref[...])
    g2_lo, g2_hi = _unpack(g2_ref[...])
    u_lo = jnp.maximum(g1_lo + g2_lo + etv[:, :half], 0.0)
    u_hi = jnp.maximum(g1_hi + g2_hi + etv[:, half:], 0.0)
    t = (jnp.sum(u_lo * g2w_ref[:, :half], axis=1, keepdims=True)
         + jnp.sum(u_hi * g2w_ref[:, half:], axis=1, keepdims=True)
         + g2b_ref[...])
    s = jnp.maximum(t, 0.0) * ew_ref[...]
    g3_lo, g3_hi = _unpack(g3_ref[...])
    out_ref[0] = g3_lo * s
    out_ref[1] = g3_hi * s


def _tc_b(g1, g2, g3, et_oh, ew, edge_emb, wet, g1b, g2w, g2b, blk):
    e, d2 = g1.shape
    o = 2 * d2
    ed = edge_emb.shape[1]
    grid = e // blk
    full = lambda i: (0, 0)
    return pl.pallas_call(
        _tcb_body,
        grid=(grid,),
        in_specs=[
            pl.BlockSpec((blk, d2), lambda i: (i, 0)),
            pl.BlockSpec((blk, d2), lambda i: (i, 0)),
            pl.BlockSpec((blk, d2), lambda i: (i, 0)),
            pl.BlockSpec((blk, 16), lambda i: (i, 0)),
            pl.BlockSpec((blk, 1), lambda i: (i, 0)),
            pl.BlockSpec((16, ed), full),
            pl.BlockSpec((ed, o), full),
            pl.BlockSpec((1, o), full),
            pl.BlockSpec((1, o), full),
            pl.BlockSpec((1, 1), full),
        ],
        out_specs=[pl.BlockSpec((2, blk, d2), lambda i: (0, i, 0))],
        out_shape=[jax.ShapeDtypeStruct((2, e, d2), jnp.float32)],
    )(g1, g2, g3, et_oh, ew, edge_emb, wet, g1b.reshape(1, o),
      g2w.reshape(1, o), g2b.reshape(1, 1))[0]


def _tcc_body(h_ref, a_ref, out_ref):
    half = h_ref.shape[1] // 2
    v0 = h_ref[:, :half] + a_ref[0]
    v1 = h_ref[:, half:] + a_ref[1]
    out_ref[:, :half] = jnp.where(v0 >= 0.0, v0, 0.01 * v0)
    out_ref[:, half:] = jnp.where(v1 >= 0.0, v1, 0.01 * v1)


def _tc_c(h, aggr, blk):
    n, o = h.shape
    grid = n // blk
    return pl.pallas_call(
        _tcc_body,
        grid=(grid,),
        in_specs=[
            pl.BlockSpec((blk, o), lambda i: (i, 0)),
            pl.BlockSpec((2, blk, o // 2), lambda i: (0, i, 0)),
        ],
        out_specs=pl.BlockSpec((blk, o), lambda i: (i, 0)),
        out_shape=jax.ShapeDtypeStruct((n, o), jnp.float32),
    )(h, aggr)


def _sc_gather(pp, qp, hp, dst2, src2):
    n, d = pp.shape
    e = dst2.shape[1]
    w = 128
    nc = 2
    steps_per_core = e // w // nc
    mesh = plsc.VectorSubcoreMesh(core_axis_name="c", subcore_axis_name="s")
    outs = [jax.ShapeDtypeStruct((e, d), jnp.float32)] * 3

    @functools.partial(pl.kernel, mesh=mesh, out_type=outs,
                       scratch_types=[pltpu.SemaphoreType.DMA,
                                      pltpu.SemaphoreType.DMA,
                                      pltpu.SemaphoreType.DMA])
    def k(p_hbm, q_hbm, h_hbm, dst_hbm, src_hbm, g1_hbm, g2_hbm, g3_hbm,
          sem1, sem2, sem3):
        idx_spec = pl.BlockSpec((1, w), lambda c, j: (0, c * steps_per_core + j))
        row_spec = pl.BlockSpec((w, d), lambda c, j: (c * steps_per_core + j, 0))

        def body(di_v, si_v, o1_v, o2_v, o3_v):
            c1 = pltpu.async_copy(p_hbm.at[di_v.at[0]], o1_v, sem1)
            c2 = pltpu.async_copy(q_hbm.at[si_v.at[0]], o2_v, sem2)
            c3 = pltpu.async_copy(h_hbm.at[si_v.at[0]], o3_v, sem3)
            c1.wait()
            c2.wait()
            c3.wait()

        pltpu.emit_pipeline(
            body,
            grid=(nc, steps_per_core),
            in_specs=[idx_spec, idx_spec],
            out_specs=[row_spec] * 3,
            core_axis_name=("c", "s"),
            dimension_semantics=(pltpu.PARALLEL, pltpu.PARALLEL),
        )(dst_hbm, src_hbm, g1_hbm, g2_hbm, g3_hbm)

    return k(pp, qp, hp, dst2, src2)


def _sc_scatter(msg, dst2, zeros_half):
    _, e, d2 = msg.shape
    n = zeros_half.shape[0]
    w = 128
    steps = e // w
    ns = 16
    rows = n // ns
    r_lo = (rows // 8) * 8
    r_hi = n - r_lo * (ns - 1)
    mesh = plsc.VectorSubcoreMesh(core_axis_name="c", subcore_axis_name="s")

    @functools.partial(
        pl.kernel, mesh=mesh,
        out_type=jax.ShapeDtypeStruct((2, n, d2), jnp.float32),
        scratch_types=[pltpu.VMEM_SHARED((n, d2), jnp.float32)])
    def k(msg_hbm, dst_hbm, z_hbm, out_hbm, aggr_sh):
        cid = lax.axis_index("c")
        sid = lax.axis_index("s")

        @pl.when(sid < ns - 1)
        def _():
            pltpu.sync_copy(z_hbm.at[pl.ds(sid * r_lo, r_lo)],
                            aggr_sh.at[pl.ds(sid * r_lo, r_lo)])

        @pl.when(sid == ns - 1)
        def _():
            pltpu.sync_copy(z_hbm.at[pl.ds((ns - 1) * r_lo, r_hi)],
                            aggr_sh.at[pl.ds((ns - 1) * r_lo, r_hi)])

        plsc.subcore_barrier()

        def body(msg_v, idx_v):
            pltpu.sync_copy(msg_v, aggr_sh.at[idx_v.at[0]], add=True)

        pltpu.emit_pipeline(
            body,
            grid=(steps,),
            in_specs=[
                pl.BlockSpec((w, d2), lambda i: (i, 0)),
                pl.BlockSpec((1, w), lambda i: (0, i)),
            ],
            out_specs=[],
            core_axis_name="s",
            dimension_semantics=(pltpu.PARALLEL,),
        )(msg_hbm.at[cid], dst_hbm)

        plsc.subcore_barrier()

        @pl.when(sid < ns - 1)
        def _():
            pltpu.sync_copy(aggr_sh.at[pl.ds(sid * r_lo, r_lo)],
                            out_hbm.at[cid].at[pl.ds(sid * r_lo, r_lo)])

        @pl.when(sid == ns - 1)
        def _():
            pltpu.sync_copy(aggr_sh.at[pl.ds((ns - 1) * r_lo, r_hi)],
                            out_hbm.at[cid].at[pl.ds((ns - 1) * r_lo, r_hi)])

    return k(msg, dst2, zeros_half)


def kernel(x, edge_index, edge_attr, node_type, edge_type, node_emb, edge_emb,
           fm_w0, fm_b0, g1_w0, g1_b0, g2_w0, g2_b0,
           fm_w1, fm_b1, g1_w1, g1_b1, g2_w1, g2_b1):
    n, f = x.shape[1], x.shape[2]
    e = edge_index.shape[1]
    o = fm_w0.shape[1]
    nd = node_emb.shape[1]

    xs = x.reshape(n, f)
    src = edge_index[0]
    dst = edge_index[1]
    nt_oh = jax.nn.one_hot(node_type, 16, dtype=jnp.float32)
    et_oh = jax.nn.one_hot(edge_type, 16, dtype=jnp.float32)
    ew = edge_attr.reshape(e, 1)
    dst2 = dst.reshape(1, e)
    src2 = src.reshape(1, e)
    zeros_half = jnp.zeros((n, o // 2), jnp.float32)

    def layer(h, pp, qp, hp, g1b, g2w, g2b, wet):
        g1, g2, g3 = _sc_gather(pp, qp, hp, dst2, src2)
        msg = _tc_b(g1, g2, g3, et_oh, ew, edge_emb, wet, g1b, g2w, g2b,
                    blk=2000)
        return _sc_scatter(msg, dst2, zeros_half)

    def wparts(g1w):
        return (g1w[0:o], g1w[o:2 * o], g1w[2 * o:2 * o + nd],
                g1w[2 * o + nd:2 * o + 2 * nd], g1w[2 * o + 2 * nd:])

    wxi0, wxj0, wnti0, wntj0, wet0 = wparts(g1_w0)
    wxi1, wxj1, wnti1, wntj1, wet1 = wparts(g1_w1)

    h1, pp1, qp1, hp1 = _tc_a(xs, fm_w0, fm_b0, wxi0, wxj0, nt_oh, node_emb,
                              wnti0, wntj0, blk=1000)
    aggr1 = layer(h1, pp1, qp1, hp1, g1_b0, g2_w0, g2_b0, wet0)
    h2, pp2, qp2, hp2 = _tc_ca(h1, aggr1, fm_w1, fm_b1, wxi1, wxj1, nt_oh,
                               node_emb, wnti1, wntj1, blk=1000)
    aggr2 = layer(h2, pp2, qp2, hp2, g1_b1, g2_w1, g2_b1, wet1)
    out = _tc_c(h2, aggr2, blk=1000)
    return out.reshape(1, n, o)


# speedup vs baseline: 1.6930x; 1.3811x over previous
import functools

import jax
import jax.numpy as jnp
from jax import lax
from jax.experimental import pallas as pl
from jax.experimental.pallas import tpu as pltpu
from jax.experimental.pallas import tpu_sc as plsc

_PREC = lax.Precision.HIGHEST


def _pack(v):
    half = v.shape[1] // 2
    lo = v[:, :half].astype(jnp.bfloat16).astype(jnp.float32)
    hi = v[:, half:].astype(jnp.bfloat16).astype(jnp.float32)
    lo_u = jax.lax.bitcast_convert_type(lo, jnp.uint32)
    hi_u = jax.lax.bitcast_convert_type(hi, jnp.uint32)
    return jax.lax.bitcast_convert_type(hi_u | (lo_u >> 16), jnp.float32)


def _unpack(w):
    w_u = jax.lax.bitcast_convert_type(w, jnp.uint32)
    lo = jax.lax.bitcast_convert_type(w_u << 16, jnp.float32)
    hi = jax.lax.bitcast_convert_type(w_u & jnp.uint32(0xFFFF0000),
                                      jnp.float32)
    return lo, hi


def _node_terms(h, oh, nemb_ref, wxi_ref, wxj_ref, wnti_ref, wntj_ref):
    ti = jnp.dot(nemb_ref[...], wnti_ref[...],
                 preferred_element_type=jnp.float32, precision=_PREC)
    tj = jnp.dot(nemb_ref[...], wntj_ref[...],
                 preferred_element_type=jnp.float32, precision=_PREC)
    p = (jnp.dot(h, wxi_ref[...], preferred_element_type=jnp.float32,
                 precision=_PREC)
         + jnp.dot(oh, ti, preferred_element_type=jnp.float32,
                   precision=_PREC))
    q = (jnp.dot(h, wxj_ref[...], preferred_element_type=jnp.float32,
                 precision=_PREC)
         + jnp.dot(oh, tj, preferred_element_type=jnp.float32,
                   precision=_PREC))
    return p, q


def _tca_body(x_ref, fmw_ref, fmb_ref, wxi_ref, wxj_ref, oh_ref, nemb_ref,
              wnti_ref, wntj_ref, h_ref, pp_ref, qp_ref, hp_ref):
    h = jnp.dot(x_ref[...], fmw_ref[...], preferred_element_type=jnp.float32,
                precision=_PREC) + fmb_ref[...]
    h_ref[...] = h
    p, q = _node_terms(h, oh_ref[...], nemb_ref, wxi_ref, wxj_ref,
                       wnti_ref, wntj_ref)
    pp_ref[...] = _pack(p)
    qp_ref[...] = _pack(q)
    hp_ref[...] = _pack(h)


def _tc_a(x, fm_w, fm_b, wxi, wxj, nt_oh, node_emb, wnti, wntj, blk):
    n, f = x.shape
    o = fm_w.shape[1]
    nd = node_emb.shape[1]
    grid = n // blk
    full = lambda i: (0, 0)
    outs = ([jax.ShapeDtypeStruct((n, o), jnp.float32)]
            + [jax.ShapeDtypeStruct((n, o // 2), jnp.float32)] * 3)
    return pl.pallas_call(
        _tca_body,
        grid=(grid,),
        in_specs=[
            pl.BlockSpec((blk, f), lambda i: (i, 0)),
            pl.BlockSpec((f, o), full),
            pl.BlockSpec((1, o), full),
            pl.BlockSpec((o, o), full),
            pl.BlockSpec((o, o), full),
            pl.BlockSpec((blk, 16), lambda i: (i, 0)),
            pl.BlockSpec((16, nd), full),
            pl.BlockSpec((nd, o), full),
            pl.BlockSpec((nd, o), full),
        ],
        out_specs=([pl.BlockSpec((blk, o), lambda i: (i, 0))]
                   + [pl.BlockSpec((blk, o // 2), lambda i: (i, 0))] * 3),
        out_shape=outs,
    )(x, fm_w, fm_b.reshape(1, o), wxi, wxj, nt_oh, node_emb, wnti, wntj)


def _tcca_body(hprev_ref, a_ref, fmw_ref, fmb_ref, wxi_ref, wxj_ref, oh_ref,
               nemb_ref, wnti_ref, wntj_ref, h_ref, pp_ref, qp_ref, hp_ref):
    half = hprev_ref.shape[1] // 2
    v0 = hprev_ref[:, :half] + a_ref[0]
    v1 = hprev_ref[:, half:] + a_ref[1]
    x = jnp.concatenate([jnp.where(v0 >= 0.0, v0, 0.01 * v0),
                         jnp.where(v1 >= 0.0, v1, 0.01 * v1)], axis=1)
    h = jnp.dot(x, fmw_ref[...], preferred_element_type=jnp.float32,
                precision=_PREC) + fmb_ref[...]
    h_ref[...] = h
    p, q = _node_terms(h, oh_ref[...], nemb_ref, wxi_ref, wxj_ref,
                       wnti_ref, wntj_ref)
    pp_ref[...] = _pack(p)
    qp_ref[...] = _pack(q)
    hp_ref[...] = _pack(h)


def _tc_ca(h_prev, aggr, fm_w, fm_b, wxi, wxj, nt_oh, node_emb, wnti, wntj,
           blk):
    n, f = h_prev.shape
    o = fm_w.shape[1]
    nd = node_emb.shape[1]
    grid = n // blk
    full = lambda i: (0, 0)
    outs = ([jax.ShapeDtypeStruct((n, o), jnp.float32)]
            + [jax.ShapeDtypeStruct((n, o // 2), jnp.float32)] * 3)
    return pl.pallas_call(
        _tcca_body,
        grid=(grid,),
        in_specs=[
            pl.BlockSpec((blk, f), lambda i: (i, 0)),
            pl.BlockSpec((2, blk, f // 2), lambda i: (0, i, 0)),
            pl.BlockSpec((f, o), full),
            pl.BlockSpec((1, o), full),
            pl.BlockSpec((o, o), full),
            pl.BlockSpec((o, o), full),
            pl.BlockSpec((blk, 16), lambda i: (i, 0)),
            pl.BlockSpec((16, nd), full),
            pl.BlockSpec((nd, o), full),
            pl.BlockSpec((nd, o), full),
        ],
        out_specs=([pl.BlockSpec((blk, o), lambda i: (i, 0))]
                   + [pl.BlockSpec((blk, o // 2), lambda i: (i, 0))] * 3),
        out_shape=outs,
    )(h_prev, aggr, fm_w, fm_b.reshape(1, o), wxi, wxj, nt_oh, node_emb,
      wnti, wntj)


def _tcb_body(g1_ref, g2_ref, g3_ref, eoh_ref, ew_ref, eemb_ref, wet_ref,
              g1b_ref, g2wl_ref, g2wh_ref, g2b_ref, out_ref):
    half = g2wl_ref.shape[0]
    et_tab = jnp.dot(eemb_ref[...], wet_ref[...],
                     preferred_element_type=jnp.float32,
                     precision=_PREC) + g1b_ref[...]
    etv = jnp.dot(eoh_ref[...], et_tab, preferred_element_type=jnp.float32)
    g1_lo, g1_hi = _unpack(g1_ref[...])
    g2_lo, g2_hi = _unpack(g2_ref[...])
    u_lo = jnp.maximum(g1_lo + g2_lo + etv[:, :half], 0.0)
    u_hi = jnp.maximum(g1_hi + g2_hi + etv[:, half:], 0.0)
    t = (jnp.dot(u_lo, g2wl_ref[...], preferred_element_type=jnp.float32)
         + jnp.dot(u_hi, g2wh_ref[...], preferred_element_type=jnp.float32)
         + g2b_ref[...])
    s = jnp.maximum(t, 0.0) * ew_ref[...]
    g3_lo, g3_hi = _unpack(g3_ref[...])
    out_ref[0] = g3_lo * s
    out_ref[1] = g3_hi * s


def _tc_b(g1, g2, g3, et_oh, ew, edge_emb, wet, g1b, g2w, g2b, blk):
    e, d2 = g1.shape
    o = 2 * d2
    ed = edge_emb.shape[1]
    grid = e // blk
    full = lambda i: (0, 0)
    return pl.pallas_call(
        _tcb_body,
        grid=(grid,),
        in_specs=[
            pl.BlockSpec((blk, d2), lambda i: (i, 0)),
            pl.BlockSpec((blk, d2), lambda i: (i, 0)),
            pl.BlockSpec((blk, d2), lambda i: (i, 0)),
            pl.BlockSpec((blk, 16), lambda i: (i, 0)),
            pl.BlockSpec((blk, 1), lambda i: (i, 0)),
            pl.BlockSpec((16, ed), full),
            pl.BlockSpec((ed, o), full),
            pl.BlockSpec((1, o), full),
            pl.BlockSpec((d2, 1), full),
            pl.BlockSpec((d2, 1), full),
            pl.BlockSpec((1, 1), full),
        ],
        out_specs=[pl.BlockSpec((2, blk, d2), lambda i: (0, i, 0))],
        out_shape=[jax.ShapeDtypeStruct((2, e, d2), jnp.float32)],
    )(g1, g2, g3, et_oh, ew, edge_emb, wet, g1b.reshape(1, o),
      g2w[:d2], g2w[d2:], g2b.reshape(1, 1))[0]


def _tcc_body(h_ref, a_ref, out_ref):
    half = h_ref.shape[1] // 2
    v0 = h_ref[:, :half] + a_ref[0]
    v1 = h_ref[:, half:] + a_ref[1]
    out_ref[:, :half] = jnp.where(v0 >= 0.0, v0, 0.01 * v0)
    out_ref[:, half:] = jnp.where(v1 >= 0.0, v1, 0.01 * v1)


def _tc_c(h, aggr, blk):
    n, o = h.shape
    grid = n // blk
    return pl.pallas_call(
        _tcc_body,
        grid=(grid,),
        in_specs=[
            pl.BlockSpec((blk, o), lambda i: (i, 0)),
            pl.BlockSpec((2, blk, o // 2), lambda i: (0, i, 0)),
        ],
        out_specs=pl.BlockSpec((blk, o), lambda i: (i, 0)),
        out_shape=jax.ShapeDtypeStruct((n, o), jnp.float32),
    )(h, aggr)


def _sc_gather(pp, qp, hp, dst2, src2):
    n, d = pp.shape
    e = dst2.shape[1]
    w = 128
    nc = 2
    steps_per_core = e // w // nc
    mesh = plsc.VectorSubcoreMesh(core_axis_name="c", subcore_axis_name="s")
    outs = [jax.ShapeDtypeStruct((e, d), jnp.float32)] * 3

    @functools.partial(pl.kernel, mesh=mesh, out_type=outs,
                       scratch_types=[pltpu.SemaphoreType.DMA,
                                      pltpu.SemaphoreType.DMA,
                                      pltpu.SemaphoreType.DMA])
    def k(p_hbm, q_hbm, h_hbm, dst_hbm, src_hbm, g1_hbm, g2_hbm, g3_hbm,
          sem1, sem2, sem3):
        idx_spec = pl.BlockSpec((1, w), lambda c, j: (0, c * steps_per_core + j))
        row_spec = pl.BlockSpec((w, d), lambda c, j: (c * steps_per_core + j, 0))

        def body(di_v, si_v, o1_v, o2_v, o3_v):
            c1 = pltpu.async_copy(p_hbm.at[di_v.at[0]], o1_v, sem1)
            c2 = pltpu.async_copy(q_hbm.at[si_v.at[0]], o2_v, sem2)
            c3 = pltpu.async_copy(h_hbm.at[si_v.at[0]], o3_v, sem3)
            c1.wait()
            c2.wait()
            c3.wait()

        pltpu.emit_pipeline(
            body,
            grid=(nc, steps_per_core),
            in_specs=[idx_spec, idx_spec],
            out_specs=[row_spec] * 3,
            core_axis_name=("c", "s"),
            dimension_semantics=(pltpu.PARALLEL, pltpu.PARALLEL),
        )(dst_hbm, src_hbm, g1_hbm, g2_hbm, g3_hbm)

    return k(pp, qp, hp, dst2, src2)


def _sc_scatter(msg, dst2, zeros_half):
    _, e, d2 = msg.shape
    n = zeros_half.shape[0]
    w = 128
    steps = e // w
    ns = 16
    rows = n // ns
    r_lo = (rows // 8) * 8
    r_hi = n - r_lo * (ns - 1)
    mesh = plsc.VectorSubcoreMesh(core_axis_name="c", subcore_axis_name="s")

    @functools.partial(
        pl.kernel, mesh=mesh,
        out_type=jax.ShapeDtypeStruct((2, n, d2), jnp.float32),
        scratch_types=[pltpu.VMEM_SHARED((n, d2), jnp.float32)])
    def k(msg_hbm, dst_hbm, z_hbm, out_hbm, aggr_sh):
        cid = lax.axis_index("c")
        sid = lax.axis_index("s")

        @pl.when(sid < ns - 1)
        def _():
            pltpu.sync_copy(z_hbm.at[pl.ds(sid * r_lo, r_lo)],
                            aggr_sh.at[pl.ds(sid * r_lo, r_lo)])

        @pl.when(sid == ns - 1)
        def _():
            pltpu.sync_copy(z_hbm.at[pl.ds((ns - 1) * r_lo, r_hi)],
                            aggr_sh.at[pl.ds((ns - 1) * r_lo, r_hi)])

        plsc.subcore_barrier()

        def body(msg_v, idx_v):
            pltpu.sync_copy(msg_v, aggr_sh.at[idx_v.at[0]], add=True)

        pltpu.emit_pipeline(
            body,
            grid=(steps,),
            in_specs=[
                pl.BlockSpec((w, d2), lambda i: (i, 0)),
                pl.BlockSpec((1, w), lambda i: (0, i)),
            ],
            out_specs=[],
            core_axis_name="s",
            dimension_semantics=(pltpu.PARALLEL,),
        )(msg_hbm.at[cid], dst_hbm)

        plsc.subcore_barrier()

        @pl.when(sid < ns - 1)
        def _():
            pltpu.sync_copy(aggr_sh.at[pl.ds(sid * r_lo, r_lo)],
                            out_hbm.at[cid].at[pl.ds(sid * r_lo, r_lo)])

        @pl.when(sid == ns - 1)
        def _():
            pltpu.sync_copy(aggr_sh.at[pl.ds((ns - 1) * r_lo, r_hi)],
                            out_hbm.at[cid].at[pl.ds((ns - 1) * r_lo, r_hi)])

    return k(msg, dst2, zeros_half)


def kernel(x, edge_index, edge_attr, node_type, edge_type, node_emb, edge_emb,
           fm_w0, fm_b0, g1_w0, g1_b0, g2_w0, g2_b0,
           fm_w1, fm_b1, g1_w1, g1_b1, g2_w1, g2_b1):
    n, f = x.shape[1], x.shape[2]
    e = edge_index.shape[1]
    o = fm_w0.shape[1]
    nd = node_emb.shape[1]

    xs = x.reshape(n, f)
    src = edge_index[0]
    dst = edge_index[1]
    nt_oh = jax.nn.one_hot(node_type, 16, dtype=jnp.float32)
    et_oh = jax.nn.one_hot(edge_type, 16, dtype=jnp.float32)
    ew = edge_attr.reshape(e, 1)
    dst2 = dst.reshape(1, e)
    src2 = src.reshape(1, e)
    zeros_half = jnp.zeros((n, o // 2), jnp.float32)

    def layer(h, pp, qp, hp, g1b, g2w, g2b, wet):
        g1, g2, g3 = _sc_gather(pp, qp, hp, dst2, src2)
        msg = _tc_b(g1, g2, g3, et_oh, ew, edge_emb, wet, g1b, g2w, g2b,
                    blk=4000)
        return _sc_scatter(msg, dst2, zeros_half)

    def wparts(g1w):
        return (g1w[0:o], g1w[o:2 * o], g1w[2 * o:2 * o + nd],
                g1w[2 * o + nd:2 * o + 2 * nd], g1w[2 * o + 2 * nd:])

    wxi0, wxj0, wnti0, wntj0, wet0 = wparts(g1_w0)
    wxi1, wxj1, wnti1, wntj1, wet1 = wparts(g1_w1)

    h1, pp1, qp1, hp1 = _tc_a(xs, fm_w0, fm_b0, wxi0, wxj0, nt_oh, node_emb,
                              wnti0, wntj0, blk=1000)
    aggr1 = layer(h1, pp1, qp1, hp1, g1_b0, g2_w0, g2_b0, wet0)
    h2, pp2, qp2, hp2 = _tc_ca(h1, aggr1, fm_w1, fm_b1, wxi1, wxj1, nt_oh,
                               node_emb, wnti1, wntj1, blk=1000)
    aggr2 = layer(h2, pp2, qp2, hp2, g1_b1, g2_w1, g2_b1, wet1)
    out = _tc_c(h2, aggr2, blk=1000)
    return out.reshape(1, n, o)
